# baseline (speedup 1.0000x reference)
import functools

import jax
import jax.numpy as jnp
from jax import lax
from jax.experimental import pallas as pl
from jax.experimental.pallas import tpu as pltpu
from jax.experimental.pallas import tpu_sc as plsc

_N_GENES = 100000
_N_DH = 64
_N_LATENT = 50
_N_OI = 4096

_SC_CORES = 2
_SC_SUBCORES = 16
_NW = _SC_CORES * _SC_SUBCORES
_ROWS_PER_W = _N_OI // _NW


def _sc_gather_body(slope_hbm, scale_hbm, shift_hbm, idx_hbm,
                    out_s, out_c, out_t,
                    idx_v, rows_s, rows_c, rows_t, sem_s, sem_c, sem_t):
    wid = lax.axis_index("s") * _SC_CORES + lax.axis_index("c")
    base = wid * _ROWS_PER_W
    pltpu.sync_copy(idx_hbm.at[pl.ds(base, _ROWS_PER_W)], idx_v)
    c1 = pltpu.async_copy(slope_hbm.at[idx_v], rows_s, sem_s)
    c2 = pltpu.async_copy(scale_hbm.at[idx_v], rows_c, sem_c)
    c3 = pltpu.async_copy(shift_hbm.at[idx_v], rows_t, sem_t)
    c1.wait()
    c2.wait()
    c3.wait()
    pltpu.sync_copy(rows_s, out_s.at[pl.ds(base, _ROWS_PER_W)])
    pltpu.sync_copy(rows_c, out_c.at[pl.ds(base, _ROWS_PER_W)])
    pltpu.sync_copy(rows_t, out_t.at[pl.ds(base, _ROWS_PER_W)])


@functools.cache
def _sc_gather():
  return pl.kernel(
    _sc_gather_body,
    out_type=[jax.ShapeDtypeStruct((_N_OI, _N_DH), jnp.float32)] * 3,
    mesh=plsc.VectorSubcoreMesh(
        core_axis_name="c", subcore_axis_name="s",
        num_cores=_SC_CORES, num_subcores=_SC_SUBCORES),
    scratch_types=[
        pltpu.VMEM((_ROWS_PER_W,), jnp.int32),
        pltpu.VMEM((_ROWS_PER_W, _N_DH), jnp.float32),
        pltpu.VMEM((_ROWS_PER_W, _N_DH), jnp.float32),
        pltpu.VMEM((_ROWS_PER_W, _N_DH), jnp.float32),
        pltpu.SemaphoreType.DMA,
        pltpu.SemaphoreType.DMA,
        pltpu.SemaphoreType.DMA,
    ],
    compiler_params=pltpu.CompilerParams(use_tc_tiling_on_sc=False),
  )


def _height_body(s_ref, c_ref, t_ref, lat_ref, o_ref):
    e = jnp.exp(c_ref[...])
    x = e * lat_ref[...] + t_ref[...]
    o_ref[...] = s_ref[...] / (1.0 + jnp.exp(-x))


_HB = 4096


def _height(sf, cf, tf, lat2):
    n = _N_OI * _N_DH
    return pl.pallas_call(
        _height_body,
        grid=(n // _HB,),
        in_specs=[
            pl.BlockSpec((_HB, 1), lambda i: (i, 0)),
            pl.BlockSpec((_HB, 1), lambda i: (i, 0)),
            pl.BlockSpec((_HB, 1), lambda i: (i, 0)),
            pl.BlockSpec((1, _N_LATENT), lambda i: (0, 0)),
        ],
        out_specs=pl.BlockSpec((_HB, _N_LATENT), lambda i: (i, 0)),
        out_shape=jax.ShapeDtypeStruct((n, _N_LATENT), jnp.float32),
    )(sf, cf, tf, lat2)


def _overall_body(w_ref, lat_ref, o_ref):
    o_ref[...] = w_ref[...] * lat_ref[...]


_OB = 10000


def _overall(w, lat2):
    return pl.pallas_call(
        _overall_body,
        grid=(_N_GENES // _OB,),
        in_specs=[
            pl.BlockSpec((_OB, 1), lambda i: (i, 0)),
            pl.BlockSpec((1, _N_LATENT), lambda i: (0, 0)),
        ],
        out_specs=pl.BlockSpec((_OB, _N_LATENT), lambda i: (i, 0)),
        out_shape=jax.ShapeDtypeStruct((_N_GENES, _N_LATENT), jnp.float32),
    )(w, lat2)


def kernel(latent, genes_oi, W_height_slope, W_height_scale, W_height_shift,
           W_overall_slope):
    g_s, g_c, g_t = _sc_gather()(
        W_height_slope, W_height_scale, W_height_shift, genes_oi)
    n = _N_OI * _N_DH
    lat2 = latent.reshape(1, _N_LATENT)
    dh = _height(g_s.reshape(n, 1), g_c.reshape(n, 1), g_t.reshape(n, 1),
                 lat2)
    do = _overall(W_overall_slope, lat2)
    return (dh.reshape(_N_OI, _N_DH, _N_LATENT),
            do.reshape(_N_GENES, 1, _N_LATENT))
